# baseline (speedup 1.0000x reference)
import functools

import jax
import jax.numpy as jnp
from jax import lax
from jax.experimental import pallas as pl
from jax.experimental.pallas import tpu as pltpu
from jax.experimental.pallas import tpu_sc as plsc


def _softplus(x):
    t = jnp.exp(-jnp.abs(x))
    s = t / (t + 2.0)
    p = (s * s) * jnp.float32(2.0 / 3.0) + jnp.float32(2.0)
    return jnp.maximum(x, 0.0) + s * p


def kernel(inputs, center_weight, delta_weight):
    B, H = inputs.shape
    V, D = center_weight.shape

    info = plsc.get_sparse_core_info()
    NC, NS = info.num_cores, info.num_subcores
    NW = NC * NS
    nb_w = B // NW
    NCH = H
    CH = nb_w
    NBUF = 3
    LN = 16
    NJ = D // LN

    idx = inputs.reshape(NW, nb_w, H).transpose(0, 2, 1).reshape(-1)
    idx = idx.astype(jnp.int32)
    mesh = plsc.VectorSubcoreMesh(core_axis_name="c", subcore_axis_name="s")

    @functools.partial(
        pl.kernel,
        out_type=jax.ShapeDtypeStruct((2, H, B, D), jnp.float32),
        mesh=mesh,
        compiler_params=pltpu.CompilerParams(use_tc_tiling_on_sc=True),
        scratch_types=(
            [pltpu.VMEM((NCH * CH,), jnp.int32)]
            + [pltpu.VMEM((CH, D), jnp.float32) for _ in range(2 * NBUF)]
            + [pltpu.SemaphoreType.DMA for _ in range(2 * NBUF)]
        ),
    )
    def run(idx_hbm, cw_hbm, dw_hbm, out_hbm, idx_v, *rest):
        cbufs = rest[0:NBUF]
        dbufs = rest[NBUF:2 * NBUF]
        gsems = rest[2 * NBUF:3 * NBUF]
        osems = rest[3 * NBUF:4 * NBUF]

        wid = lax.axis_index("s") * NC + lax.axis_index("c")
        b_base = wid * nb_w

        pltpu.sync_copy(idx_hbm.at[pl.ds(wid * NCH * CH, NCH * CH)], idx_v)

        def gather_copies(g, s):
            ix = idx_v.at[pl.ds(g * CH, CH)]
            return (
                pltpu.make_async_copy(cw_hbm.at[ix], cbufs[s], gsems[s]),
                pltpu.make_async_copy(dw_hbm.at[ix], dbufs[s], gsems[s]),
            )

        def out_copies(g, s):
            return (
                pltpu.make_async_copy(
                    cbufs[s], out_hbm.at[0, g, pl.ds(b_base, nb_w)], osems[s]),
                pltpu.make_async_copy(
                    dbufs[s], out_hbm.at[1, g, pl.ds(b_base, nb_w)], osems[s]),
            )

        def compute(s):
            cb, db = cbufs[s], dbufs[s]

            def row(i, carry):
                for u in range(4):
                    r = i * 4 + u
                    for j in range(NJ):
                        sl = pl.ds(j * LN, LN)
                        c = cb[r, sl]
                        sp = _softplus(db[r, sl])
                        cb[r, sl] = c - sp
                        db[r, sl] = c + sp
                return carry

            lax.fori_loop(0, CH // 4, row, 0)

        for g in range(min(NBUF - 1, NCH)):
            for cp in gather_copies(g, g % NBUF):
                cp.start()

        def chunk_step(g, s, wait_prev_out, fire_next):
            for cp in gather_copies(g, s):
                cp.wait()
            compute(s)
            for cp in out_copies(g, s):
                cp.start()
            if fire_next:
                sp_ = (s + NBUF - 1) % NBUF
                if wait_prev_out:
                    for cp in out_copies(g - 1, sp_):
                        cp.wait()
                for cp in gather_copies(g + NBUF - 1, sp_):
                    cp.start()

        for g in range(NBUF):
            chunk_step(g, g % NBUF, wait_prev_out=(g >= 1), fire_next=True)

        main_turns = (NCH - NBUF - 2) // NBUF

        def turn(i, carry):
            for s2 in range(NBUF):
                chunk_step(i * NBUF + s2, s2, wait_prev_out=True, fire_next=True)
            return carry

        lax.fori_loop(1, 1 + main_turns, turn, 0)

        for g in range(NBUF + main_turns * NBUF, NCH):
            chunk_step(g, g % NBUF, wait_prev_out=False,
                       fire_next=(g + NBUF - 1 < NCH))

        for g in range(max(0, NCH - NBUF), NCH):
            for cp in out_copies(g, g % NBUF):
                cp.wait()

    out = run(idx, center_weight, delta_weight)
    return jnp.transpose(out, (0, 2, 1, 3))


# speedup vs baseline: 1.3440x; 1.3440x over previous
import functools

import jax
import jax.numpy as jnp
from jax import lax
from jax.experimental import pallas as pl
from jax.experimental.pallas import tpu as pltpu
from jax.experimental.pallas import tpu_sc as plsc


def _softplus(x):
    t = jnp.exp(-jnp.abs(x))
    s = t / (t + 2.0)
    p = (s * s) * jnp.float32(2.0 / 3.0) + jnp.float32(2.0)
    return jnp.maximum(x, 0.0) + s * p


def kernel(inputs, center_weight, delta_weight):
    B, H = inputs.shape
    V, D = center_weight.shape

    info = plsc.get_sparse_core_info()
    NC, NS = info.num_cores, info.num_subcores
    NW = NC * NS
    nb_w = B // NW
    NCH = H
    CH = nb_w
    NBUF = 3
    LN = 16
    NJ = D // LN

    idx = inputs.reshape(NW, nb_w, H).transpose(0, 2, 1).reshape(-1)
    idx = idx.astype(jnp.int32)
    mesh = plsc.VectorSubcoreMesh(core_axis_name="c", subcore_axis_name="s")

    @functools.partial(
        pl.kernel,
        out_type=jax.ShapeDtypeStruct((2, H, B, D), jnp.float32),
        mesh=mesh,
        compiler_params=pltpu.CompilerParams(use_tc_tiling_on_sc=True),
        scratch_types=(
            [pltpu.VMEM((NCH * CH,), jnp.int32)]
            + [pltpu.VMEM((CH, D), jnp.float32) for _ in range(2 * NBUF)]
            + [pltpu.SemaphoreType.DMA for _ in range(2 * NBUF)]
        ),
    )
    def run(idx_hbm, cw_hbm, dw_hbm, out_hbm, idx_v, *rest):
        cbufs = rest[0:NBUF]
        dbufs = rest[NBUF:2 * NBUF]
        gsems = rest[2 * NBUF:3 * NBUF]
        osems = rest[3 * NBUF:4 * NBUF]

        wid = lax.axis_index("s") * NC + lax.axis_index("c")
        b_base = wid * nb_w

        pltpu.sync_copy(idx_hbm.at[pl.ds(wid * NCH * CH, NCH * CH)], idx_v)

        def gather_copies(g, s):
            ix = idx_v.at[pl.ds(g * CH, CH)]
            return (
                pltpu.make_async_copy(cw_hbm.at[ix], cbufs[s], gsems[s]),
                pltpu.make_async_copy(dw_hbm.at[ix], dbufs[s], gsems[s]),
            )

        def out_copies(g, s):
            return (
                pltpu.make_async_copy(
                    cbufs[s], out_hbm.at[0, g, pl.ds(b_base, nb_w)], osems[s]),
                pltpu.make_async_copy(
                    dbufs[s], out_hbm.at[1, g, pl.ds(b_base, nb_w)], osems[s]),
            )

        def compute(s):
            cb, db = cbufs[s], dbufs[s]

            def row(i, carry):
                for u in range(2):
                    r = i * 2 + u
                    for j in range(NJ):
                        sl = pl.ds(j * LN, LN)
                        c = cb[r, sl]
                        sp = _softplus(db[r, sl])
                        cb[r, sl] = c - sp
                        db[r, sl] = c + sp
                return carry

            lax.fori_loop(0, CH // 2, row, 0)

        for g in range(min(NBUF - 1, NCH)):
            for cp in gather_copies(g, g % NBUF):
                cp.start()

        def chunk_step(g, s, wait_prev_out, fire_next):
            for cp in gather_copies(g, s):
                cp.wait()
            compute(s)
            for cp in out_copies(g, s):
                cp.start()
            if fire_next:
                sp_ = (s + NBUF - 1) % NBUF
                if wait_prev_out:
                    for cp in out_copies(g - 1, sp_):
                        cp.wait()
                for cp in gather_copies(g + NBUF - 1, sp_):
                    cp.start()

        for g in range(NBUF):
            chunk_step(g, g % NBUF, wait_prev_out=(g >= 1), fire_next=True)

        main_turns = (NCH - NBUF - 2) // NBUF

        def turn(i, carry):
            for s2 in range(NBUF):
                chunk_step(i * NBUF + s2, s2, wait_prev_out=True, fire_next=True)
            return carry

        lax.fori_loop(1, 1 + main_turns, turn, 0)

        for g in range(NBUF + main_turns * NBUF, NCH):
            chunk_step(g, g % NBUF, wait_prev_out=False,
                       fire_next=(g + NBUF - 1 < NCH))

        for g in range(max(0, NCH - NBUF), NCH):
            for cp in out_copies(g, g % NBUF):
                cp.wait()

    out = run(idx, center_weight, delta_weight)
    return jnp.transpose(out, (0, 2, 1, 3))
